# Initial kernel scaffold; baseline (speedup 1.0000x reference)
import functools

import jax
import jax.numpy as jnp
from jax import lax
from jax.experimental import pallas as pl
from jax.experimental.pallas import tpu as pltpu
from jax.experimental.pallas import tpu_sc as plsc

NC = 2
NS = 16
NW = NC * NS
LANE = 128
BLKS = 4
CHUNK = LANE * BLKS


def _sc_aggregate(D, n_rows, n_trash, rows2d, with_counts):
    n_tot = n_rows + n_trash
    assert n_tot % NS == 0 and n_rows % NS == 0
    zrows = n_tot // NS
    orows = n_rows // NS
    assert zrows % 8 == 0
    zbuf_r = zrows // 8
    assert rows2d % (NW * BLKS) == 0
    w_rows = rows2d // NW
    n_chunks = w_rows // BLKS

    out_type = [jax.ShapeDtypeStruct((NC, n_rows, D), jnp.float32)]
    scratch = [
        pltpu.VMEM_SHARED((n_tot, D), jnp.float32),
        pltpu.VMEM((BLKS, LANE), jnp.int32),
        pltpu.VMEM((BLKS, LANE), jnp.int32),
        pltpu.VMEM((CHUNK, D), jnp.float32),
        pltpu.VMEM((zbuf_r, D), jnp.float32),
        pltpu.SemaphoreType.DMA,
    ]
    if with_counts:
        out_type.append(jax.ShapeDtypeStruct((NC, n_rows, 16), jnp.float32))
        scratch += [
            pltpu.VMEM_SHARED((n_tot, 16), jnp.float32),
            pltpu.VMEM((zbuf_r, 16), jnp.float32),
            pltpu.VMEM((LANE, 16), jnp.float32),
        ]

    mesh = plsc.VectorSubcoreMesh(
        core_axis_name="c", subcore_axis_name="s", num_cores=NC,
        num_subcores=NS)

    def body(x_hbm, src_hbm, dst_hbm, *refs):
        if with_counts:
            (out_hbm, cnt_hbm, agg_sh, sidx, didx, rows, zrow, sem,
             cnt_sh, zcnt, ones) = refs
        else:
            out_hbm, agg_sh, sidx, didx, rows, zrow, sem = refs
        cid = lax.axis_index("c")
        sid = lax.axis_index("s")
        wid = cid * NS + sid

        zv = jnp.zeros((16,), jnp.float32)

        def zfill(i, _):
            for j in range(D // 16):
                zrow[i, pl.ds(j * 16, 16)] = zv
            return 0
        lax.fori_loop(0, zbuf_r, zfill, 0)
        if with_counts:
            ov = jnp.ones((16,), jnp.float32)

            def czfill(i, _):
                zcnt[i, :] = zv
                return 0
            lax.fori_loop(0, zbuf_r, czfill, 0)

            def ofill(i, _):
                ones[i, :] = ov
                return 0
            lax.fori_loop(0, LANE, ofill, 0)

        for g in range(8):
            pltpu.sync_copy(
                zrow, agg_sh.at[pl.ds(sid * zrows + g * zbuf_r, zbuf_r)])
            if with_counts:
                pltpu.sync_copy(
                    zcnt, cnt_sh.at[pl.ds(sid * zrows + g * zbuf_r, zbuf_r)])
        plsc.subcore_barrier()

        row0 = wid * w_rows

        def chunk_body(c, _):
            base = row0 + c * BLKS
            pltpu.sync_copy(src_hbm.at[pl.ds(base, BLKS)], sidx)
            pltpu.sync_copy(dst_hbm.at[pl.ds(base, BLKS)], didx)
            cps = [
                pltpu.async_copy(x_hbm.at[sidx.at[j]],
                                 rows.at[pl.ds(j * LANE, LANE)], sem)
                for j in range(BLKS)
            ]
            for cp in cps:
                cp.wait()
            for j in range(BLKS):
                pltpu.sync_copy(rows.at[pl.ds(j * LANE, LANE)],
                                agg_sh.at[didx.at[j]], add=True)
            if with_counts:
                for j in range(BLKS):
                    pltpu.sync_copy(ones, cnt_sh.at[didx.at[j]], add=True)
            return 0
        lax.fori_loop(0, n_chunks, chunk_body, 0)
        plsc.subcore_barrier()

        pltpu.sync_copy(agg_sh.at[pl.ds(sid * orows, orows)],
                        out_hbm.at[cid, pl.ds(sid * orows, orows)])
        if with_counts:
            pltpu.sync_copy(cnt_sh.at[pl.ds(sid * orows, orows)],
                            cnt_hbm.at[cid, pl.ds(sid * orows, orows)])

    return pl.kernel(body, out_type=out_type, mesh=mesh,
                     scratch_types=scratch)


def _tc_layer(relu, p_ref, pcnt_ref, x_ref, wl_ref, b_ref, wr_ref, g_ref,
              be_ref, o_ref):
    cnt = pcnt_ref[0, :, 0:1] + pcnt_ref[1, :, 0:1]
    rinv = 1.0 / jnp.maximum(cnt, 1.0)
    agg = (p_ref[0] + p_ref[1]) * rinv
    h = lax.dot_general(agg, wl_ref[...], (((1,), (1,)), ((), ())),
                        preferred_element_type=jnp.float32)
    h = h + lax.dot_general(x_ref[...], wr_ref[...],
                            (((1,), (1,)), ((), ())),
                            preferred_element_type=jnp.float32)
    h = h + b_ref[...]
    m = jnp.mean(h, axis=0, keepdims=True)
    d = h - m
    v = jnp.mean(d * d, axis=0, keepdims=True)
    h = d * lax.rsqrt(v + 1e-5) * g_ref[...] + be_ref[...]
    if relu:
        h = jnp.maximum(h, 0.0)
    o_ref[...] = h


def _dense_layer(p, pcnt, x, W_l, b, W_r, gamma, beta, relu):
    n, d = x.shape
    return pl.pallas_call(
        functools.partial(_tc_layer, relu),
        out_shape=jax.ShapeDtypeStruct((n, d), jnp.float32),
    )(p, pcnt, x, W_l, b.reshape(1, d), W_r, gamma.reshape(1, d),
      beta.reshape(1, d))


def kernel(x, edge_index, W1_l, b1, W1_r, gamma1, beta1, W2_l, b2, W2_r,
           gamma2, beta2):
    n, d = x.shape
    e = edge_index.shape[1]
    e_pad = -(-e // (NW * CHUNK)) * (NW * CHUNK)
    pad = e_pad - e
    n_trash = 112
    assert (n + n_trash) % (NS * 8) == 0

    src = edge_index[0]
    dst = edge_index[1]
    if pad:
        fill = jnp.arange(pad, dtype=jnp.int32)
        src = jnp.concatenate([src, fill % n])
        dst = jnp.concatenate([dst, n + fill % n_trash])
    src2d = src.reshape(-1, LANE)
    dst2d = dst.reshape(-1, LANE)
    rows2d = src2d.shape[0]

    agg1 = _sc_aggregate(d, n, n_trash, rows2d, True)
    agg2 = _sc_aggregate(d, n, n_trash, rows2d, False)

    p1, pcnt = agg1(x, src2d, dst2d)
    h = _dense_layer(p1, pcnt, x, W1_l, b1, W1_r, gamma1, beta1, True)
    p2 = agg2(h, src2d, dst2d)
    if isinstance(p2, (list, tuple)):
        p2 = p2[0]
    return _dense_layer(p2, pcnt, h, W2_l, b2, W2_r, gamma2, beta2, False)


# speedup vs baseline: 4.1362x; 4.1362x over previous
import functools

import jax
import jax.numpy as jnp
from jax import lax
from jax.experimental import pallas as pl
from jax.experimental.pallas import tpu as pltpu
from jax.experimental.pallas import tpu_sc as plsc

NC = 2
NS = 16
NW = NC * NS
LANE = 64
NTRASH = 240
QR = 8
OC = 48


def _mesh():
    return plsc.VectorSubcoreMesh(
        core_axis_name="c", subcore_axis_name="s", num_cores=NC,
        num_subcores=NS)


def _sc_aggregate(D, n_rows, rows2d):
    n_tot = n_rows + NTRASH
    zrows = n_tot // NS
    assert zrows % (2 * LANE) == 0
    or8 = (n_rows // (NS * 8)) * 8
    otail = n_rows - NS * or8
    assert or8 % OC == 0 and otail % 8 == 0
    assert rows2d % (NW * QR) == 0
    w_rows = rows2d // NW
    n_stage = w_rows // QR

    def body(x_hbm, src_hbm, dst_hbm, out_hbm, agg_sh, sidx, didx, rows,
             sem0, sem1):
        sems = (sem0, sem1)
        cid = lax.axis_index("c")
        sid = lax.axis_index("s")
        wid = cid * NS + sid
        w_edges = w_rows * LANE

        zv = jnp.zeros((16,), jnp.float32)
        for i in range(2 * LANE):
            for j in range(D // 16):
                rows[i, pl.ds(j * 16, 16)] = zv

        z0 = sid * zrows

        def zero_body(g, _):
            pltpu.sync_copy(rows, agg_sh.at[pl.ds(z0 + g * 2 * LANE,
                                                  2 * LANE)])
            return 0
        lax.fori_loop(0, zrows // (2 * LANE), zero_body, 0)
        plsc.subcore_barrier()

        def blk_body(blk, _):
            e0 = wid * w_edges + blk * LANE
            pltpu.sync_copy(src_hbm.at[pl.ds(e0, LANE)], sidx)
            pltpu.sync_copy(dst_hbm.at[pl.ds(e0, LANE)], didx)
            slot = rows.at[pl.ds(0, LANE)]
            pltpu.async_copy(x_hbm.at[sidx], slot, sems[0]).wait()
            pltpu.sync_copy(slot, agg_sh.at[didx], add=True)
            return 0
        lax.fori_loop(0, w_rows, blk_body, 0)
        plsc.subcore_barrier()

        def out_body(g, _):
            o0 = sid * or8 + g * OC
            pltpu.sync_copy(agg_sh.at[pl.ds(o0, OC)],
                            out_hbm.at[cid, pl.ds(o0, OC)])
            return 0
        lax.fori_loop(0, or8 // OC, out_body, 0)
        if otail:
            @pl.when(sid == 0)
            def _():
                pltpu.sync_copy(agg_sh.at[pl.ds(NS * or8, otail)],
                                out_hbm.at[cid, pl.ds(NS * or8, otail)])

    return pl.kernel(
        body,
        out_type=jax.ShapeDtypeStruct((NC, n_rows, D), jnp.float32),
        mesh=_mesh(),
        scratch_types=[
            pltpu.VMEM_SHARED((n_tot, D), jnp.float32),
            pltpu.VMEM((LANE,), jnp.int32),
            pltpu.VMEM((LANE,), jnp.int32),
            pltpu.VMEM((2 * LANE, D), jnp.float32),
            pltpu.SemaphoreType.DMA,
            pltpu.SemaphoreType.DMA,
        ])


def _sc_counts(n_rows, rows2d):
    n_tot = n_rows + NTRASH
    zrows = n_tot // NS
    assert zrows % LANE == 0
    or8 = (n_rows // (NS * 8)) * 8
    otail = n_rows - NS * or8
    w_rows = rows2d // NW
    D = 128

    def body(dst_hbm, cnt_hbm, cnt_sh, didx, ones, zbuf):
        cid = lax.axis_index("c")
        sid = lax.axis_index("s")
        wid = cid * NS + sid
        w_edges = w_rows * LANE

        zv = jnp.zeros((16,), jnp.float32)
        ov = jnp.ones((16,), jnp.float32)
        for i in range(LANE):
            for j in range(D // 16):
                ones[i, pl.ds(j * 16, 16)] = ov
                zbuf[i, pl.ds(j * 16, 16)] = zv

        z0 = sid * zrows

        def zero_body(g, _):
            pltpu.sync_copy(zbuf, cnt_sh.at[pl.ds(z0 + g * LANE, LANE)])
            return 0
        lax.fori_loop(0, zrows // LANE, zero_body, 0)
        plsc.subcore_barrier()

        def blk_body(blk, _):
            e0 = wid * w_edges + blk * LANE
            pltpu.sync_copy(dst_hbm.at[pl.ds(e0, LANE)], didx)
            pltpu.sync_copy(ones, cnt_sh.at[didx], add=True)
            return 0
        lax.fori_loop(0, w_rows, blk_body, 0)
        plsc.subcore_barrier()

        def out_body(g, _):
            o0 = sid * or8 + g * OC
            pltpu.sync_copy(cnt_sh.at[pl.ds(o0, OC)],
                            cnt_hbm.at[cid, pl.ds(o0, OC)])
            return 0
        lax.fori_loop(0, or8 // OC, out_body, 0)
        if otail:
            @pl.when(sid == 0)
            def _():
                pltpu.sync_copy(cnt_sh.at[pl.ds(NS * or8, otail)],
                                cnt_hbm.at[cid, pl.ds(NS * or8, otail)])

    return pl.kernel(
        body,
        out_type=jax.ShapeDtypeStruct((NC, n_rows, 128), jnp.float32),
        mesh=_mesh(),
        scratch_types=[
            pltpu.VMEM_SHARED((n_tot, 128), jnp.float32),
            pltpu.VMEM((LANE,), jnp.int32),
            pltpu.VMEM((LANE, 128), jnp.float32),
            pltpu.VMEM((LANE, 128), jnp.float32),
        ])


def _tc_layer(relu, p_ref, pcnt_ref, x_ref, wl_ref, b_ref, wr_ref, g_ref,
              be_ref, o_ref):
    cnt = pcnt_ref[0, :, 0:1] + pcnt_ref[1, :, 0:1]
    rinv = 1.0 / jnp.maximum(cnt, 1.0)
    agg = (p_ref[0] + p_ref[1]) * rinv
    h = lax.dot_general(agg, wl_ref[...], (((1,), (1,)), ((), ())),
                        preferred_element_type=jnp.float32)
    h = h + lax.dot_general(x_ref[...], wr_ref[...],
                            (((1,), (1,)), ((), ())),
                            preferred_element_type=jnp.float32)
    h = h + b_ref[...]
    m = jnp.mean(h, axis=0, keepdims=True)
    d = h - m
    v = jnp.mean(d * d, axis=0, keepdims=True)
    h = d * lax.rsqrt(v + 1e-5) * g_ref[...] + be_ref[...]
    if relu:
        h = jnp.maximum(h, 0.0)
    o_ref[...] = h


def _dense_layer(p, pcnt, x, W_l, b, W_r, gamma, beta, relu):
    n, d = x.shape
    return pl.pallas_call(
        functools.partial(_tc_layer, relu),
        out_shape=jax.ShapeDtypeStruct((n, d), jnp.float32),
    )(p, pcnt, x, W_l, b.reshape(1, d), W_r, gamma.reshape(1, d),
      beta.reshape(1, d))


def kernel(x, edge_index, W1_l, b1, W1_r, gamma1, beta1, W2_l, b2, W2_r,
           gamma2, beta2):
    n, d = x.shape
    e = edge_index.shape[1]
    grain = NW * LANE * QR
    e_pad = -(-e // grain) * grain
    pad = e_pad - e

    src = edge_index[0]
    dst = edge_index[1]
    if pad:
        fill = jnp.arange(pad, dtype=jnp.int32)
        src = jnp.concatenate([src, fill % n])
        dst = jnp.concatenate([dst, n + fill % NTRASH])
    rows2d = src.shape[0] // LANE

    agg = _sc_aggregate(d, n, rows2d)
    pcnt = _sc_counts(n, rows2d)(dst)
    p1 = agg(x, src, dst)
    h = _dense_layer(p1, pcnt, x, W1_l, b1, W1_r, gamma1, beta1, True)
    p2 = agg(h, src, dst)
    return _dense_layer(p2, pcnt, h, W2_l, b2, W2_r, gamma2, beta2, False)
